# baseline (speedup 1.0000x reference)
import functools

import jax
import jax.numpy as jnp
from jax import lax
from jax.experimental import pallas as pl
from jax.experimental.pallas import tpu as pltpu
from jax.experimental.pallas import tpu_sc as plsc

_HID = 4096
_K = 64
_TOP_K = 8
_TB = 1024

_NC = 2
_NS = 16
_NW = _NC * _NS
_CH = 256


def _mm_body(h_ref, wt_ref, b_ref, z_ref):
    z_ref[...] = (
        jnp.dot(h_ref[...], wt_ref[...], preferred_element_type=jnp.float32)
        + b_ref[...]
    )


def _project(hf, wt, bias, row0, rows):
    nb = rows // _TB
    off = row0 // _TB
    return pl.pallas_call(
        _mm_body,
        grid=(nb,),
        in_specs=[
            pl.BlockSpec((_TB, _HID), lambda i: (i + off, 0)),
            pl.BlockSpec((_HID, 2 * _K), lambda i: (0, 0)),
            pl.BlockSpec((1, 2 * _K), lambda i: (0, 0)),
        ],
        out_specs=pl.BlockSpec((_TB, 2 * _K), lambda i: (i, 0)),
        out_shape=jax.ShapeDtypeStruct((rows, 2 * _K), jnp.float32),
    )(hf, wt, bias)


def _merge_desc(va, pa, vb, pb):
    vbr = lax.rev(vb, (0,))
    pbr = lax.rev(pb, (0,))
    take_a = va >= vbr
    v = jnp.where(take_a, va, vbr)
    p = jnp.where(take_a, pa, pbr)
    return plsc.sort_key_val(v, p, descending=True)


def _sc_gate_body(z_hbm, o_hbm, zbuf, obuf, *, tpw, ch):
    wid = lax.axis_index("s") * _NC + lax.axis_index("c")
    base = wid * tpw
    i16 = lax.iota(jnp.int32, 16)
    lane8 = i16 < _TOP_K

    for c in range(tpw // ch):
        pltpu.sync_copy(z_hbm.at[pl.ds(base + c * ch, ch), :], zbuf)

        @plsc.parallel_loop(0, ch, 1, unroll=2)
        def tok_body(t):
            parts = []
            for j in range(4):
                g = zbuf[t, pl.ds(16 * j, 16)]
                parts.append(plsc.sort_key_val(g, i16 + 16 * j, descending=True))
            v01, p01 = _merge_desc(*parts[0], *parts[1])
            v23, p23 = _merge_desc(*parts[2], *parts[3])
            v, p = _merge_desc(v01, p01, v23, p23)
            m = jnp.max(v)
            e = jnp.where(lane8, jnp.exp(v - m), jnp.float32(0.0))
            den = jnp.sum(e)
            row = jnp.broadcast_to(t, (16,)).astype(jnp.int32)
            ev = plsc.load_gather(zbuf, [row, p + _K])
            num = jnp.sum(e * ev)
            res = jnp.broadcast_to(num, (16,)) / jnp.broadcast_to(den, (16,))
            o = c * ch + t
            plsc.store_scatter(
                obuf,
                [jnp.broadcast_to(o, (16,)).astype(jnp.int32)],
                res,
                mask=i16 == 0,
            )

    pltpu.sync_copy(obuf, o_hbm.at[pl.ds(base, tpw)])


def _sc_gate(z):
    n_tok = z.shape[0]
    tpw = n_tok // _NW
    ch = min(_CH, tpw)
    mesh = plsc.VectorSubcoreMesh(
        core_axis_name="c", subcore_axis_name="s", num_cores=_NC,
        num_subcores=_NS,
    )
    f = functools.partial(
        pl.kernel,
        out_type=jax.ShapeDtypeStruct((n_tok,), jnp.float32),
        mesh=mesh,
        compiler_params=pltpu.CompilerParams(needs_layout_passes=False),
        scratch_types=[
            pltpu.VMEM((ch, 2 * _K), jnp.float32),
            pltpu.VMEM((tpw,), jnp.float32),
        ],
    )(functools.partial(_sc_gate_body, tpw=tpw, ch=ch))
    return f(z)


def kernel(h, W_e, b_e, W_g):
    B, L, _ = h.shape
    hf = h.reshape(B * L, _HID)
    wt = jnp.concatenate([W_g, W_e], axis=0).T
    bias = jnp.concatenate([jnp.zeros((_K,), b_e.dtype), b_e]).reshape(1, 2 * _K)
    sizes = (8192, 8192, 8192, 6144, 2048)
    outs = []
    row0 = 0
    for rows in sizes:
        z = _project(hf, wt, bias, row0, rows)
        outs.append(_sc_gate(z))
        row0 += rows
    out = jnp.concatenate(outs)
    return out.reshape(B, L)


# speedup vs baseline: 1.0315x; 1.0315x over previous
import functools

import jax
import jax.numpy as jnp
from jax import lax
from jax.experimental import pallas as pl
from jax.experimental.pallas import tpu as pltpu
from jax.experimental.pallas import tpu_sc as plsc

_HID = 4096
_K = 64
_TOP_K = 8
_TB = 1024

_NC = 2
_NS = 16
_NW = _NC * _NS
_CH = 256


def _mm_body(h_ref, wt_ref, b_ref, z_ref):
    z_ref[...] = (
        jnp.dot(h_ref[...], wt_ref[...], preferred_element_type=jnp.float32)
        + b_ref[...]
    )


def _project(hf, wt, bias, row0, rows):
    nb = rows // _TB
    off = row0 // _TB
    return pl.pallas_call(
        _mm_body,
        grid=(nb,),
        in_specs=[
            pl.BlockSpec((_TB, _HID), lambda i: (i + off, 0)),
            pl.BlockSpec((_HID, 2 * _K), lambda i: (0, 0)),
            pl.BlockSpec((1, 2 * _K), lambda i: (0, 0)),
        ],
        out_specs=pl.BlockSpec((_TB, 2 * _K), lambda i: (i, 0)),
        out_shape=jax.ShapeDtypeStruct((rows, 2 * _K), jnp.float32),
    )(hf, wt, bias)


def _merge_desc(va, pa, vb, pb):
    vbr = lax.rev(vb, (0,))
    pbr = lax.rev(pb, (0,))
    take_a = va >= vbr
    v = jnp.where(take_a, va, vbr)
    p = jnp.where(take_a, pa, pbr)
    return plsc.sort_key_val(v, p, descending=True)


def _sc_gate_body(z_hbm, o_hbm, zbuf, obuf, *, tpw, ch):
    wid = lax.axis_index("s") * _NC + lax.axis_index("c")
    base = wid * tpw
    i16 = lax.iota(jnp.int32, 16)
    lane8 = i16 < _TOP_K

    for c in range(tpw // ch):
        pltpu.sync_copy(z_hbm.at[pl.ds(base + c * ch, ch), :], zbuf)

        @plsc.parallel_loop(0, ch, 1, unroll=2)
        def tok_body(t):
            parts = []
            for j in range(4):
                g = zbuf[t, pl.ds(16 * j, 16)]
                parts.append(plsc.sort_key_val(g, i16 + 16 * j, descending=True))
            v01, p01 = _merge_desc(*parts[0], *parts[1])
            v23, p23 = _merge_desc(*parts[2], *parts[3])
            v, p = _merge_desc(v01, p01, v23, p23)
            m = jnp.max(v)
            e = jnp.where(lane8, jnp.exp(v - m), jnp.float32(0.0))
            den = jnp.sum(e)
            row = jnp.broadcast_to(t, (16,)).astype(jnp.int32)
            ev = plsc.load_gather(zbuf, [row, p + _K])
            num = jnp.sum(e * ev)
            res = jnp.broadcast_to(num, (16,)) / jnp.broadcast_to(den, (16,))
            o = c * ch + t
            plsc.store_scatter(
                obuf,
                [jnp.broadcast_to(o, (16,)).astype(jnp.int32)],
                res,
                mask=i16 == 0,
            )

    pltpu.sync_copy(obuf, o_hbm.at[pl.ds(base, tpw)])


def _sc_gate(z):
    n_tok = z.shape[0]
    tpw = n_tok // _NW
    ch = min(_CH, tpw)
    mesh = plsc.VectorSubcoreMesh(
        core_axis_name="c", subcore_axis_name="s", num_cores=_NC,
        num_subcores=_NS,
    )
    f = functools.partial(
        pl.kernel,
        out_type=jax.ShapeDtypeStruct((n_tok,), jnp.float32),
        mesh=mesh,
        compiler_params=pltpu.CompilerParams(needs_layout_passes=False),
        scratch_types=[
            pltpu.VMEM((ch, 2 * _K), jnp.float32),
            pltpu.VMEM((tpw,), jnp.float32),
        ],
    )(functools.partial(_sc_gate_body, tpw=tpw, ch=ch))
    return f(z)


def kernel(h, W_e, b_e, W_g):
    B, L, _ = h.shape
    hf = h.reshape(B * L, _HID)
    wt = jnp.concatenate([W_g, W_e], axis=0).T
    bias = jnp.concatenate([jnp.zeros((_K,), b_e.dtype), b_e]).reshape(1, 2 * _K)
    sizes = (8192, 8192, 8192, 8192)
    outs = []
    row0 = 0
    for rows in sizes:
        z = _project(hf, wt, bias, row0, rows)
        outs.append(_sc_gate(z))
        row0 += rows
    out = jnp.concatenate(outs)
    return out.reshape(B, L)
